# Initial kernel scaffold; baseline (speedup 1.0000x reference)
import jax
import jax.numpy as jnp
from jax.experimental import pallas as pl


def kernel(x, edge_index, W1, b1, W2, b2):
    raise NotImplementedError("write your pallas kernel here")


# speedup vs baseline: 11.6044x; 11.6044x over previous
import functools

import jax
import jax.numpy as jnp
import numpy as np
from jax import lax
from jax.experimental import pallas as pl
from jax.experimental.pallas import tpu as pltpu
from jax.experimental.pallas import tpu_sc as plsc

N = 10000
D = 128
NC = 2
NS = 16
K = 128

RPT = 632
NACC = NS * RPT
DEGW = 128

_mesh = plsc.VectorSubcoreMesh(core_axis_name="c", subcore_axis_name="s")


def _make_deg_kernel(chunks):
    @functools.partial(
        pl.kernel,
        out_type=jax.ShapeDtypeStruct((NC, NACC, DEGW), jnp.float32),
        mesh=_mesh,
        scratch_types=[
            pltpu.VMEM((K,), jnp.int32),
            pltpu.VMEM((K, DEGW), jnp.float32),
            pltpu.VMEM_SHARED((NACC, DEGW), jnp.float32),
            pltpu.SemaphoreType.DMA,
        ],
    )
    def deg_kernel(dst_hbm, zeros_hbm, ones_hbm, out_hbm, didx_v, ones_v,
                   acc_s, sem):
        cid = lax.axis_index("c")
        sid = lax.axis_index("s")
        wid = cid * NS + sid
        pltpu.sync_copy(zeros_hbm.at[pl.ds(sid * RPT, RPT)],
                        acc_s.at[pl.ds(sid * RPT, RPT)])
        pltpu.sync_copy(ones_hbm, ones_v)
        plsc.subcore_barrier()
        base = wid * chunks * K

        @pl.loop(jnp.int32(0), jnp.int32(chunks * K), step=jnp.int32(K))
        def _chunk(off0):
            o = pl.multiple_of(base + off0, K)
            pltpu.async_copy(dst_hbm.at[pl.ds(o, K)], didx_v, sem).wait()
            pltpu.sync_copy(ones_v, acc_s.at[didx_v], add=True)
        plsc.subcore_barrier()
        pltpu.sync_copy(acc_s.at[pl.ds(sid * RPT, RPT)],
                        out_hbm.at[cid, pl.ds(sid * RPT, RPT)])

    return deg_kernel


def _make_scatter_kernel(chunks):
    @functools.partial(
        pl.kernel,
        out_type=jax.ShapeDtypeStruct((NC, NACC, D), jnp.float32),
        mesh=_mesh,
        scratch_types=[
            pltpu.VMEM((K,), jnp.int32),
            pltpu.VMEM((K,), jnp.int32),
            pltpu.VMEM((K, D), jnp.float32),
            pltpu.VMEM_SHARED((NACC, D), jnp.float32),
            pltpu.SemaphoreType.DMA,
            pltpu.SemaphoreType.DMA,
        ],
    )
    def scatter_kernel(g_hbm, src_hbm, dst_hbm, zeros_hbm, out_hbm,
                       sidx_v, didx_v, rows_v, acc_s, sem0, sem1):
        cid = lax.axis_index("c")
        sid = lax.axis_index("s")
        wid = cid * NS + sid
        pltpu.sync_copy(zeros_hbm.at[pl.ds(sid * RPT, RPT)],
                        acc_s.at[pl.ds(sid * RPT, RPT)])
        plsc.subcore_barrier()
        base = wid * chunks * K

        @pl.loop(jnp.int32(0), jnp.int32(chunks * K), step=jnp.int32(K))
        def _chunk(off0):
            o = pl.multiple_of(base + off0, K)
            c0 = pltpu.async_copy(src_hbm.at[pl.ds(o, K)], sidx_v, sem0)
            c1 = pltpu.async_copy(dst_hbm.at[pl.ds(o, K)], didx_v, sem1)
            c0.wait()
            c1.wait()
            pltpu.async_copy(g_hbm.at[sidx_v], rows_v, sem0).wait()
            pltpu.sync_copy(rows_v, acc_s.at[didx_v], add=True)
        plsc.subcore_barrier()
        pltpu.sync_copy(acc_s.at[pl.ds(sid * RPT, RPT)],
                        out_hbm.at[cid, pl.ds(sid * RPT, RPT)])

    return scatter_kernel


_RB = 1000


def _dinv_of(deg_ref):
    deg = deg_ref[0, :, 0:1] + deg_ref[1, :, 0:1] + 1.0
    return lax.rsqrt(deg)


def _tc1_body(x_ref, w_ref, deg_ref, o_ref):
    dinv = _dinv_of(deg_ref)
    h = jnp.dot(x_ref[...], w_ref[...], preferred_element_type=jnp.float32)
    o_ref[...] = h * dinv


def _tc2_body(g_ref, acc_ref, deg_ref, b_ref, w_ref, o_ref):
    dinv = _dinv_of(deg_ref)
    s = g_ref[...] + acc_ref[0] + acc_ref[1]
    z = jnp.maximum(dinv * s + b_ref[...], 0.0)
    h = jnp.dot(z, w_ref[...], preferred_element_type=jnp.float32)
    o_ref[...] = h * dinv


def _tc3_body(g_ref, acc_ref, deg_ref, b_ref, o_ref):
    dinv = _dinv_of(deg_ref)
    s = g_ref[...] + acc_ref[0] + acc_ref[1]
    o_ref[...] = dinv * s + b_ref[...]


_i0 = np.int32(0)
_row_spec = pl.BlockSpec((_RB, D), lambda i: (i, _i0))
_acc_spec = pl.BlockSpec((NC, _RB, D), lambda i: (_i0, i, _i0))
_deg_spec = pl.BlockSpec((NC, _RB, DEGW), lambda i: (_i0, i, _i0))
_w_spec = pl.BlockSpec((D, D), lambda i: (_i0, _i0))
_b_spec = pl.BlockSpec((1, D), lambda i: (_i0, _i0))
_grid = (N // _RB,)

_tc1 = pl.pallas_call(
    _tc1_body, grid=_grid,
    in_specs=[_row_spec, _w_spec, _deg_spec],
    out_specs=_row_spec,
    out_shape=jax.ShapeDtypeStruct((N, D), jnp.float32))

_tc2 = pl.pallas_call(
    _tc2_body, grid=_grid,
    in_specs=[_row_spec, _acc_spec, _deg_spec, _b_spec, _w_spec],
    out_specs=_row_spec,
    out_shape=jax.ShapeDtypeStruct((N, D), jnp.float32))

_tc3 = pl.pallas_call(
    _tc3_body, grid=_grid,
    in_specs=[_row_spec, _acc_spec, _deg_spec, _b_spec],
    out_specs=_row_spec,
    out_shape=jax.ShapeDtypeStruct((N, D), jnp.float32))


def kernel(x, edge_index, W1, b1, W2, b2):
    E = edge_index.shape[1]
    chunks = -(-E // (NC * NS * K))
    epad = NC * NS * K * chunks - E

    src = edge_index[0].astype(jnp.int32)
    dst = edge_index[1].astype(jnp.int32)
    if epad:
        src = jnp.concatenate([src, jnp.zeros((epad,), jnp.int32)])
        dst = jnp.concatenate([dst, jnp.full((epad,), N, jnp.int32)])

    zeros_deg = jnp.zeros((NACC, DEGW), jnp.float32)
    ones_deg = jnp.ones((K, DEGW), jnp.float32)
    zeros_acc = jnp.zeros((NACC, D), jnp.float32)

    deg_k = _make_deg_kernel(chunks)
    scat_k = _make_scatter_kernel(chunks)

    deg = deg_k(dst, zeros_deg, ones_deg)

    b1r = b1.reshape(1, D).astype(jnp.float32)
    b2r = b2.reshape(1, D).astype(jnp.float32)

    g1 = _tc1(x, W1, deg)
    acc1 = scat_k(g1, src, dst, zeros_acc)
    g2 = _tc2(g1, acc1, deg, b1r, W2)
    acc2 = scat_k(g2, src, dst, zeros_acc)
    out = _tc3(g2, acc2, deg, b2r)
    return out
